# baseline (speedup 1.0000x reference)
import functools

import jax
import jax.numpy as jnp
from jax import lax
from jax.experimental import pallas as pl
from jax.experimental.pallas import tpu as pltpu
from jax.experimental.pallas import tpu_sc as plsc

_NC = 2
_NS = 16
_NW = _NC * _NS

_N_NODE = 10000
_N_PAD = 10240
_BAGS_PER_W = 320
_CHUNK = 32
_N_CHUNKS = _BAGS_PER_W // _CHUNK
_NODE_BAG = 8
_D = 128


def _node_sc_body(idx_hbm, tab_hbm, out_hbm, tab_sp, idx_v, rows_v, out_v, sem, sem_o):
    c_ax = lax.axis_index("c")
    s_ax = lax.axis_index("s")
    w = s_ax * _NC + c_ax

    @pl.when(s_ax == 0)
    def _():
        pltpu.sync_copy(tab_hbm, tab_sp)

    plsc.subcore_barrier()

    pltpu.sync_copy(idx_hbm.at[w], idx_v)

    def issue(k, buf):
        return [
            pltpu.async_copy(
                tab_sp.at[idx_v.at[j * _N_CHUNKS + k]],
                rows_v.at[buf, j], sem,
            )
            for j in range(_NODE_BAG)
        ]

    pending = {0: issue(0, 0), 1: None}
    out_cp = {0: None, 1: None}
    for k in range(_N_CHUNKS):
        buf = k % 2
        for cp in pending[buf]:
            cp.wait()
        if k + 1 < _N_CHUNKS:
            pending[1 - buf] = issue(k + 1, 1 - buf)
        if out_cp[buf] is not None:
            out_cp[buf].wait()

        def cbody(b, carry, buf=buf):
            for c in range(_D // 16):
                acc = rows_v[buf, 0, b, pl.ds(c * 16, 16)]
                for j in range(1, _NODE_BAG):
                    acc = acc + rows_v[buf, j, b, pl.ds(c * 16, 16)]
                out_v[buf, b, pl.ds(c * 16, 16)] = acc
            return carry

        lax.fori_loop(0, _CHUNK, cbody, 0)
        out_cp[buf] = pltpu.async_copy(
            out_v.at[buf],
            out_hbm.at[pl.ds(w * _BAGS_PER_W + k * _CHUNK, _CHUNK)],
            sem_o,
        )
    for buf in (0, 1):
        if out_cp[buf] is not None:
            out_cp[buf].wait()


def _node_sc_call(node_feats, node_table):
    pad = jnp.zeros((_N_PAD - _N_NODE, _NODE_BAG), node_feats.dtype)
    idxp = jnp.concatenate([node_feats, pad], axis=0)
    idx3 = (
        idxp.reshape(_NW, _BAGS_PER_W, _NODE_BAG)
        .transpose(0, 2, 1)
        .reshape(_NW, _NODE_BAG * _N_CHUNKS, _CHUNK)
    )
    mesh = plsc.VectorSubcoreMesh(
        core_axis_name="c", subcore_axis_name="s",
        num_cores=_NC, num_subcores=_NS,
    )
    f = pl.kernel(
        _node_sc_body,
        out_type=jax.ShapeDtypeStruct((_N_PAD, _D), jnp.float32),
        mesh=mesh,
        scratch_types=[
            pltpu.VMEM_SHARED((_D, _D), jnp.float32),
            pltpu.VMEM((_NODE_BAG * _N_CHUNKS, _CHUNK), jnp.int32),
            pltpu.VMEM((2, _NODE_BAG, _CHUNK, _D), jnp.float32),
            pltpu.VMEM((2, _CHUNK, _D), jnp.float32),
            pltpu.SemaphoreType.DMA,
            pltpu.SemaphoreType.DMA,
        ],
    )
    return f(idx3, node_table)[:_N_NODE]


def _edge_tc_body(idx_ref, tab_ref, out_ref, *, vocab, bag):
    idxT = idx_ref[...]
    tab = tab_ref[...]
    r = idxT.shape[1]
    iota = lax.broadcasted_iota(jnp.int32, (vocab, r), 0)
    cntT = jnp.zeros((vocab, r), jnp.bfloat16)
    for j in range(bag):
        row = lax.broadcast_in_dim(idxT[j], (vocab, r), (1,))
        cntT = cntT + (row == iota).astype(jnp.bfloat16)
    out_ref[...] = lax.dot_general(
        cntT, tab, (((0,), (0,)), ((), ())),
        preferred_element_type=jnp.float32)


def _edge_tc_call(featsT, table, block):
    bag, n = featsT.shape
    vocab, d = table.shape
    return pl.pallas_call(
        functools.partial(_edge_tc_body, vocab=vocab, bag=bag),
        grid=((n + block - 1) // block,),
        in_specs=[
            pl.BlockSpec((bag, block), lambda i: (0, i)),
            pl.BlockSpec((vocab, d), lambda i: (0, 0)),
        ],
        out_specs=pl.BlockSpec((block, d), lambda i: (i, 0)),
        out_shape=jax.ShapeDtypeStruct((n, d), jnp.float32),
    )(featsT, table.astype(jnp.bfloat16))


def kernel(node_feats, edge_feats, node_table, edge_table):
    node_out = _node_sc_call(node_feats, node_table)
    edge_out = _edge_tc_call(edge_feats.T, edge_table, 12800)
    return node_out, edge_out


# speedup vs baseline: 1.2586x; 1.2586x over previous
import functools

import jax
import jax.numpy as jnp
from jax.experimental import pallas as pl


def _bag_body(idx_ref, tab_ref, out_ref, *, vocab, bag):
    idxT = idx_ref[...]
    tab = tab_ref[...]
    r = idxT.shape[1]
    iota = jax.lax.broadcasted_iota(jnp.int32, (vocab, r), 0)
    cntT = jnp.zeros((vocab, r), jnp.bfloat16)
    for j in range(bag):
        row = jax.lax.broadcast_in_dim(idxT[j], (vocab, r), (1,))
        cntT = cntT + (row == iota).astype(jnp.bfloat16)
    out_ref[...] = jax.lax.dot_general(
        cntT, tab, (((0,), (0,)), ((), ())),
        preferred_element_type=jnp.float32)


def _bag_call(featsT, table, block):
    bag, n = featsT.shape
    vocab, d = table.shape
    return pl.pallas_call(
        functools.partial(_bag_body, vocab=vocab, bag=bag),
        grid=((n + block - 1) // block,),
        in_specs=[
            pl.BlockSpec((bag, block), lambda i: (0, i)),
            pl.BlockSpec((vocab, d), lambda i: (0, 0)),
        ],
        out_specs=pl.BlockSpec((block, d), lambda i: (i, 0)),
        out_shape=jax.ShapeDtypeStruct((n, d), jnp.float32),
    )(featsT, table.astype(jnp.bfloat16))


def kernel(node_feats, edge_feats, node_table, edge_table):
    node_out = _bag_call(node_feats.T, node_table, 5120)
    edge_out = _bag_call(edge_feats.T, edge_table, 12800)
    return node_out, edge_out
